# Initial kernel scaffold; baseline (speedup 1.0000x reference)
import jax
import jax.numpy as jnp
from jax.experimental import pallas as pl


def kernel(x, table, W1, b1, W2, b2, W3, b3):
    raise NotImplementedError("write your pallas kernel here")


# speedup vs baseline: 3.2112x; 3.2112x over previous
import functools

import jax
import jax.numpy as jnp
from jax import lax
from jax.experimental import pallas as pl
from jax.experimental.pallas import tpu as pltpu
from jax.experimental.pallas import tpu_sc as plsc

VOCAB = 100277
EMBED = 64
SEQ = 32
BATCH = 4096

N_FLAT = BATCH * SEQ
NW = 32
ROWS_PER_W = N_FLAT // NW
CH = 128
NCH = ROWS_PER_W // CH


def _sc_gather(table, idx):
  mesh = plsc.VectorSubcoreMesh(core_axis_name="c", subcore_axis_name="s")

  @functools.partial(
      pl.kernel,
      mesh=mesh,
      compiler_params=pltpu.CompilerParams(use_tc_tiling_on_sc=False),
      out_type=jax.ShapeDtypeStruct((N_FLAT, EMBED), jnp.float32),
      scratch_types=[
          pltpu.VMEM((NCH, CH), jnp.int32),
          pltpu.VMEM((CH, EMBED), jnp.float32),
          pltpu.VMEM((CH, EMBED), jnp.float32),
          pltpu.SemaphoreType.DMA,
          pltpu.SemaphoreType.DMA,
      ],
  )
  def k(table_hbm, idx_hbm, out_hbm, idx_v, buf0, buf1, sem0, sem1):
    wid = lax.axis_index("s") * 2 + lax.axis_index("c")
    base = wid * ROWS_PER_W
    pltpu.sync_copy(idx_hbm.at[wid], idx_v)

    bufs = (buf0, buf1)
    sems = (sem0, sem1)

    pltpu.async_copy(table_hbm.at[idx_v.at[0]], buf0, sem0)

    @pl.loop(0, NCH, step=2)
    def _body(j):
      for b in range(2):
        cur = j + b

        @pl.when(cur + 1 < NCH)
        def _():
          pltpu.async_copy(
              table_hbm.at[idx_v.at[cur + 1]], bufs[1 - b], sems[1 - b])

        pltpu.make_async_copy(
            table_hbm.at[idx_v.at[cur]], bufs[b], sems[b]).wait()
        pltpu.sync_copy(bufs[b], out_hbm.at[pl.ds(base + cur * CH, CH)])

  return k(table, idx)


def _tc_mlp(emb, w1t, b1, w2t, b2, w3t, b3):
  BB = 512
  OUTP = w3t.shape[1]

  def body(e_ref, w1_ref, b1_ref, w2_ref, b2_ref, w3_ref, b3_ref, o_ref):
    h = jnp.dot(e_ref[...], w1_ref[...], preferred_element_type=jnp.float32)
    h = jnp.maximum(h + b1_ref[...], 0.0)
    h = jnp.dot(h, w2_ref[...], preferred_element_type=jnp.float32)
    h = jnp.maximum(h + b2_ref[...], 0.0)
    o_ref[...] = (
        jnp.dot(h, w3_ref[...], preferred_element_type=jnp.float32)
        + b3_ref[...])

  full = lambda a: pl.BlockSpec(a.shape, lambda i: (0,) * a.ndim)
  return pl.pallas_call(
      body,
      grid=(BATCH // BB,),
      in_specs=[
          pl.BlockSpec((BB, SEQ * EMBED), lambda i: (i, 0)),
          full(w1t), full(b1), full(w2t), full(b2), full(w3t), full(b3),
      ],
      out_specs=pl.BlockSpec((BB, OUTP), lambda i: (i, 0)),
      out_shape=jax.ShapeDtypeStruct((BATCH, OUTP), jnp.float32),
  )(emb, w1t, b1, w2t, b2, w3t, b3)


def kernel(x, table, W1, b1, W2, b2, W3, b3):
  idx = x.astype(jnp.int32).reshape(NW, NCH, CH)
  emb = _sc_gather(table, idx)
  emb = emb.reshape(BATCH, SEQ * EMBED)

  nout = W3.shape[0]
  w3t = jnp.zeros((W3.shape[1], 32), jnp.float32).at[:, :nout].set(W3.T)
  b3p = jnp.zeros((1, 32), jnp.float32).at[:, :nout].set(b3[None, :])
  out = _tc_mlp(emb, W1.T, b1[None, :], W2.T, b2[None, :], w3t, b3p)
  return out[:, :nout]


# speedup vs baseline: 3.8663x; 1.2040x over previous
import functools

import jax
import jax.numpy as jnp
from jax import lax
from jax.experimental import pallas as pl
from jax.experimental.pallas import tpu as pltpu
from jax.experimental.pallas import tpu_sc as plsc

VOCAB = 100277
EMBED = 64
SEQ = 32
BATCH = 4096

N_FLAT = BATCH * SEQ
NW = 32
ROWS_PER_W = N_FLAT // NW
CH = 128
NCH = ROWS_PER_W // CH


def _sc_gather(table, idx):
  mesh = plsc.VectorSubcoreMesh(core_axis_name="c", subcore_axis_name="s")

  @functools.partial(
      pl.kernel,
      mesh=mesh,
      compiler_params=pltpu.CompilerParams(use_tc_tiling_on_sc=False),
      out_type=jax.ShapeDtypeStruct((N_FLAT, EMBED), jnp.float32),
      scratch_types=[
          pltpu.VMEM((NCH, CH), jnp.int32),
          pltpu.VMEM((CH, EMBED), jnp.float32),
          pltpu.VMEM((CH, EMBED), jnp.float32),
          pltpu.SemaphoreType.DMA,
          pltpu.SemaphoreType.DMA,
      ],
  )
  def k(table_hbm, idx_hbm, out_hbm, idx_v, buf0, buf1, sem0, sem1):
    wid = lax.axis_index("s") * 2 + lax.axis_index("c")
    base = wid * ROWS_PER_W
    pltpu.sync_copy(idx_hbm.at[wid], idx_v)

    bufs = (buf0, buf1)
    sems = (sem0, sem1)

    pltpu.async_copy(table_hbm.at[idx_v.at[0]], buf0, sem0)

    @pl.loop(0, NCH, step=2)
    def _body(j):
      for b in range(2):
        cur = j + b

        @pl.when(cur + 1 < NCH)
        def _():
          pltpu.async_copy(
              table_hbm.at[idx_v.at[cur + 1]], bufs[1 - b], sems[1 - b])

        pltpu.make_async_copy(
            table_hbm.at[idx_v.at[cur]], bufs[b], sems[b]).wait()
        pltpu.sync_copy(bufs[b], out_hbm.at[pl.ds(base + cur * CH, CH)])

  return k(table, idx)


NP = SEQ * EMBED // 128


def _tc_mlp(emb3, w1r, b1, w2t, b2, w3t, b3):
  BB = 512
  OUTP = w3t.shape[1]

  def body(e_ref, w1_ref, b1_ref, w2_ref, b2_ref, w3_ref, b3_ref, o_ref):
    h = jnp.dot(
        e_ref[:, 0, :], w1_ref[0], preferred_element_type=jnp.float32)
    for p in range(1, NP):
      h += jnp.dot(
          e_ref[:, p, :], w1_ref[p], preferred_element_type=jnp.float32)
    h = jnp.maximum(h + b1_ref[...], 0.0)
    h = jnp.dot(h, w2_ref[...], preferred_element_type=jnp.float32)
    h = jnp.maximum(h + b2_ref[...], 0.0)
    o_ref[...] = (
        jnp.dot(h, w3_ref[...], preferred_element_type=jnp.float32)
        + b3_ref[...])

  full = lambda a: pl.BlockSpec(a.shape, lambda i: (0,) * a.ndim)
  return pl.pallas_call(
      body,
      grid=(BATCH // BB,),
      in_specs=[
          pl.BlockSpec((BB, NP, 128), lambda i: (i, 0, 0)),
          full(w1r), full(b1), full(w2t), full(b2), full(w3t), full(b3),
      ],
      out_specs=pl.BlockSpec((BB, OUTP), lambda i: (i, 0)),
      out_shape=jax.ShapeDtypeStruct((BATCH, OUTP), jnp.float32),
  )(emb3, w1r, b1, w2t, b2, w3t, b3)


def kernel(x, table, W1, b1, W2, b2, W3, b3):
  idx = x.astype(jnp.int32).reshape(NW, NCH, CH)
  emb = _sc_gather(table, idx)
  emb3 = emb.reshape(BATCH, NP, 128)

  nout = W3.shape[0]
  w1r = W1.T.reshape(NP, 128, 128)
  w3t = jnp.zeros((W3.shape[1], 32), jnp.float32).at[:, :nout].set(W3.T)
  b3p = jnp.zeros((1, 32), jnp.float32).at[:, :nout].set(b3[None, :])
  out = _tc_mlp(emb3, w1r, b1[None, :], W2.T, b2[None, :], w3t, b3p)
  return out[:, :nout]
